# baseline (speedup 1.0000x reference)
import functools

import numpy as np
import jax
import jax.numpy as jnp
from jax import lax
from jax.experimental import pallas as pl
from jax.experimental.pallas import tpu as pltpu
from jax.experimental.pallas import tpu_sc as plsc

N = 1024
G = 16384
CONTEXT_LEN = 2048
M = 2
C = CONTEXT_LEN - M
CP = 2048
MAX_TOTAL_MRNA_UMIS = 10000.0
MSB = np.int32(np.uint32(0x80000000).view(np.int32))
EXP1 = np.int32(np.uint32(0x3F800000).view(np.int32))
BINOM_ITERS = 10


_U32 = np.uint32


def _np_threefry(k0, k1, x0, x1):
    k0, k1 = _U32(k0), _U32(k1)
    x0, x1 = _U32(x0), _U32(x1)
    ks = [k0, k1, k0 ^ k1 ^ _U32(0x1BD11BDA)]
    rot = [[13, 15, 26, 6], [17, 29, 16, 24]]
    x0 = _U32(x0 + ks[0])
    x1 = _U32(x1 + ks[1])
    for i in range(5):
        for r in rot[i % 2]:
            x0 = _U32(x0 + x1)
            x1 = _U32((_U32(x1 << _U32(r)) | _U32(x1 >> _U32(32 - r))))
            x1 = x0 ^ x1
        x0 = _U32(x0 + ks[(i + 1) % 3])
        x1 = _U32(x1 + ks[(i + 2) % 3] + _U32(i + 1))
    return x0, x1


def _np_fold(key, data):
    return _np_threefry(key[0], key[1], 0, data)


def _binom_subkeys():
    key = (_U32(0), _U32(7))
    subs = []
    with np.errstate(over="ignore"):
        for _ in range(BINOM_ITERS):
            subs.append(_np_fold(key, 0))
            key = _np_fold(key, 1)
    as_i32 = lambda v: int(np.asarray(v, np.uint32).view(np.int32))
    return [(as_i32(a), as_i32(b)) for a, b in subs]


_BINOM_SUBKEYS = _binom_subkeys()


def _rotl(x, r):
    return jnp.bitwise_or(
        lax.shift_left(x, jnp.int32(r)),
        lax.shift_right_logical(x, jnp.int32(32 - r)),
    )


def _tf_bits(k0, k1, ctr):
    ks2 = jnp.bitwise_xor(jnp.bitwise_xor(k0, k1), jnp.int32(0x1BD11BDA))
    ks = [k0, k1, ks2]
    rot = [[13, 15, 26, 6], [17, 29, 16, 24]]
    x0 = jnp.broadcast_to(ks[0], ctr.shape)
    x1 = ctr + ks[1]
    for i in range(5):
        for r in rot[i % 2]:
            x0 = x0 + x1
            x1 = _rotl(x1, r)
            x1 = jnp.bitwise_xor(x0, x1)
        x0 = x0 + ks[(i + 1) % 3]
        x1 = x1 + ks[(i + 2) % 3] + jnp.int32(i + 1)
    return jnp.bitwise_xor(x0, x1)


def _partner(x, d, m_first):
    b, g = x.shape
    if 2 * d == g:
        return jnp.roll(x, d, axis=-1)
    if 2 * d >= 512:
        return jnp.roll(x.reshape(b, g // (2 * d), 2 * d), d, axis=-1).reshape(b, g)
    return jnp.where(m_first, jnp.roll(x, -d, axis=-1), jnp.roll(x, d, axis=-1))


def _bitonic_sort(key, val, n, iota1):
    s = 2
    while s <= n:
        d = s // 2
        while d >= 1:
            m_first = (iota1 & d) == 0
            kp = _partner(key, d, m_first)
            vp = _partner(val, d, m_first)
            gt = (key > kp) | ((key == kp) & (val > vp))
            mdir = ((iota1 & d) != 0) != ((iota1 & s) != 0)
            take_p = gt != mdir
            key = jnp.where(take_p, kp, key)
            val = jnp.where(take_p, vp, val)
            d //= 2
        s *= 2
    return key, val


def _sort_body(keys_ref, x1_ref, pos_ref):
    ks = keys_ref[0, 0]
    r = ks.shape[0]
    iota2 = lax.broadcasted_iota(jnp.int32, (2 * r, G), 1)
    iota1 = lax.broadcasted_iota(jnp.int32, (1, G), 1)
    k0 = jnp.concatenate([ks[:, 0:1], ks[:, 2:3]], axis=0)
    k1 = jnp.concatenate([ks[:, 1:2], ks[:, 3:4]], axis=0)
    bits = _tf_bits(k0, k1, iota2)
    skey = jnp.bitwise_xor(bits, MSB)
    _, val = _bitonic_sort(skey, iota2, G, iota1)
    x1_ref[...] = val[:r]
    pos_ref[...] = val[r:, :CP]


def _binom_body(gv_ref, prm_ref, out_ref):
    gv = gv_ref[...]
    prm = prm_ref[0, 0]
    r = gv.shape[0]
    lognm = prm[:, 0:1]
    plt = prm[:, 1:2]
    count = jnp.floor(gv)
    row0 = pl.program_id(0) * r
    iota_c = lax.broadcasted_iota(jnp.int32, (r, CP), 1)
    iota_r = lax.broadcasted_iota(jnp.int32, (r, CP), 0)
    ctr = (row0 + iota_r) * jnp.int32(C) + iota_c
    num_geom = jnp.zeros((r, CP), jnp.float32)
    gsum = jnp.zeros((r, CP), jnp.float32)
    for t in range(BINOM_ITERS):
        k0, k1 = _BINOM_SUBKEYS[t]
        bits = _tf_bits(jnp.int32(k0), jnp.int32(k1), ctr)
        fb = jnp.bitwise_or(lax.shift_right_logical(bits, jnp.int32(9)), EXP1)
        u = lax.bitcast_convert_type(fb, jnp.float32) - jnp.float32(1.0)
        num_geom = jnp.where(gsum <= count, num_geom + 1.0, num_geom)
        geom = jnp.ceil(jnp.log(u) / lognm)
        gsum = gsum + geom
    samples = num_geom - jnp.float32(1.0)
    out_ref[...] = jnp.where(plt > 0.5, samples, count - samples)


_NW = 32
_ROWS_PER_W = N // _NW


@functools.lru_cache(maxsize=1)
def _make_sc_gather():
    mesh = plsc.VectorSubcoreMesh(core_axis_name="c", subcore_axis_name="s")

    @functools.partial(
        pl.kernel,
        mesh=mesh,
        compiler_params=pltpu.CompilerParams(needs_layout_passes=False),
        out_type=[
            jax.ShapeDtypeStruct((N, CP), jnp.int32),
            jax.ShapeDtypeStruct((N, CP), jnp.float32),
        ],
        scratch_types=[
            pltpu.VMEM((G,), jnp.int32),
            pltpu.VMEM((G,), jnp.float32),
            pltpu.VMEM((CP,), jnp.int32),
            pltpu.VMEM((CP,), jnp.int32),
            pltpu.VMEM((CP,), jnp.float32),
        ],
    )
    def _sc_gather(x1_hbm, gv_hbm, pos_hbm, perm_hbm, out_hbm,
                   x1_v, gv_v, pos_v, perm_v, val_v):
        wid = lax.axis_index("s") * 2 + lax.axis_index("c")

        def row_body(i, _):
            row = wid * _ROWS_PER_W + i
            pltpu.sync_copy(x1_hbm.at[row], x1_v)
            pltpu.sync_copy(gv_hbm.at[row], gv_v)
            pltpu.sync_copy(pos_hbm.at[row], pos_v)

            def g_body(j, _):
                pos16 = pos_v[pl.ds(j * 16, 16)]
                perm16 = plsc.load_gather(x1_v, [pos16])
                val16 = plsc.load_gather(gv_v, [perm16])
                perm_v[pl.ds(j * 16, 16)] = perm16
                val_v[pl.ds(j * 16, 16)] = val16
                return 0

            lax.fori_loop(0, CP // 16, g_body, 0)
            pltpu.sync_copy(perm_v, perm_hbm.at[row])
            pltpu.sync_copy(val_v, out_hbm.at[row])
            return 0

        lax.fori_loop(0, _ROWS_PER_W, row_body, 0)

    return _sc_gather


def _jnp_threefry(k0, k1, x0, x1):
    i32 = lambda v: jnp.asarray(v, jnp.int32)
    k0, k1, x0, x1 = i32(k0), i32(k1), i32(x0), i32(x1)
    ks = [k0, k1, k0 ^ k1 ^ jnp.int32(0x1BD11BDA)]
    rot = [[13, 15, 26, 6], [17, 29, 16, 24]]
    x0 = x0 + ks[0]
    x1 = x1 + ks[1]
    for i in range(5):
        for r in rot[i % 2]:
            x0 = x0 + x1
            x1 = _rotl(x1, r)
            x1 = x0 ^ x1
        x0 = x0 + ks[(i + 1) % 3]
        x1 = x1 + ks[(i + 2) % 3] + jnp.int32(i + 1)
    return x0, x1


def _row_sort_keys(obs_seed_n):
    seed = obs_seed_n.astype(jnp.int32)
    z = jnp.zeros_like(seed)
    rk0, rk1 = _jnp_threefry(jnp.int32(0), jnp.int32(42), z, seed)
    s10, s11 = _jnp_threefry(rk0, rk1, z, z + 1)
    c10, c11 = _jnp_threefry(rk0, rk1, z, z)
    s20, s21 = _jnp_threefry(c10, c11, z, z + 1)
    return jnp.stack([s10, s11, s20, s21], axis=-1)


SORT_R = 8
BIN_R = 64


def kernel(gene_value_ng, total_mrna_umis_ng, assay_n, cell_type_n, tissue_n,
           gene_id_g, obs_seed_n):
    n, g = gene_value_ng.shape

    keys_n4 = _row_sort_keys(obs_seed_n).reshape(n // SORT_R, 1, SORT_R, 4)

    x1_ng, pos_ncp = pl.pallas_call(
        _sort_body,
        grid=(n // SORT_R,),
        compiler_params=pltpu.CompilerParams(dimension_semantics=("parallel",)),
        in_specs=[pl.BlockSpec((1, 1, SORT_R, 4), lambda i: (i, 0, 0, 0))],
        out_specs=[
            pl.BlockSpec((SORT_R, G), lambda i: (i, 0)),
            pl.BlockSpec((SORT_R, CP), lambda i: (i, 0)),
        ],
        out_shape=[
            jax.ShapeDtypeStruct((n, G), jnp.int32),
            jax.ShapeDtypeStruct((n, CP), jnp.int32),
        ],
    )(keys_n4)

    perm_ncp, gval_ncp = _make_sc_gather()(x1_ng, gene_value_ng, pos_ncp)

    total_n1 = total_mrna_umis_ng[:, 0:1].astype(jnp.float32)
    dtot_n1 = jnp.minimum(total_n1, MAX_TOTAL_MRNA_UMIS)
    p_n1 = dtot_n1 / total_n1
    plt_n1 = (p_n1 < 0.5).astype(jnp.float32)
    q_n1 = jnp.where(p_n1 < 0.5, p_n1, 1.0 - p_n1)
    lognm_n1 = jnp.log1p(-q_n1)
    prm = jnp.concatenate([lognm_n1, plt_n1], axis=-1).reshape(n // BIN_R, 1, BIN_R, 2)

    sampled_ncp = pl.pallas_call(
        _binom_body,
        grid=(n // BIN_R,),
        compiler_params=pltpu.CompilerParams(dimension_semantics=("parallel",)),
        in_specs=[
            pl.BlockSpec((BIN_R, CP), lambda i: (i, 0)),
            pl.BlockSpec((1, 1, BIN_R, 2), lambda i: (i, 0, 0, 0)),
        ],
        out_specs=pl.BlockSpec((BIN_R, CP), lambda i: (i, 0)),
        out_shape=jax.ShapeDtypeStruct((n, CP), jnp.float32),
    )(gval_ncp, prm)

    sampled_nc = sampled_ncp[:, :C]
    gene_id_nc = perm_ncp[:, :C]
    rounded_total_nc = jnp.broadcast_to(jnp.round(dtot_n1), (n, C))
    assay_nc = jnp.broadcast_to(assay_n[:, None], (n, C)).astype(jnp.int32)
    return (
        sampled_nc,
        rounded_total_nc,
        gene_id_nc,
        assay_nc,
        cell_type_n.astype(jnp.int32),
        tissue_n.astype(jnp.int32),
    )


# speedup vs baseline: 1.1662x; 1.1662x over previous
import functools

import numpy as np
import jax
import jax.numpy as jnp
from jax import lax
from jax.experimental import pallas as pl
from jax.experimental.pallas import tpu as pltpu
from jax.experimental.pallas import tpu_sc as plsc

N = 1024
G = 16384
CONTEXT_LEN = 2048
M = 2
C = CONTEXT_LEN - M
CP = 2048
MAX_TOTAL_MRNA_UMIS = 10000.0
MSB = np.int32(np.uint32(0x80000000).view(np.int32))
EXP1 = np.int32(np.uint32(0x3F800000).view(np.int32))
BINOM_ITERS = 10


_U32 = np.uint32


def _np_threefry(k0, k1, x0, x1):
    k0, k1 = _U32(k0), _U32(k1)
    x0, x1 = _U32(x0), _U32(x1)
    ks = [k0, k1, k0 ^ k1 ^ _U32(0x1BD11BDA)]
    rot = [[13, 15, 26, 6], [17, 29, 16, 24]]
    x0 = _U32(x0 + ks[0])
    x1 = _U32(x1 + ks[1])
    for i in range(5):
        for r in rot[i % 2]:
            x0 = _U32(x0 + x1)
            x1 = _U32((_U32(x1 << _U32(r)) | _U32(x1 >> _U32(32 - r))))
            x1 = x0 ^ x1
        x0 = _U32(x0 + ks[(i + 1) % 3])
        x1 = _U32(x1 + ks[(i + 2) % 3] + _U32(i + 1))
    return x0, x1


def _np_fold(key, data):
    return _np_threefry(key[0], key[1], 0, data)


def _binom_subkeys():
    key = (_U32(0), _U32(7))
    subs = []
    with np.errstate(over="ignore"):
        for _ in range(BINOM_ITERS):
            subs.append(_np_fold(key, 0))
            key = _np_fold(key, 1)
    as_i32 = lambda v: int(np.asarray(v, np.uint32).view(np.int32))
    return [(as_i32(a), as_i32(b)) for a, b in subs]


_BINOM_SUBKEYS = _binom_subkeys()


def _rotl(x, r):
    return jnp.bitwise_or(
        lax.shift_left(x, jnp.int32(r)),
        lax.shift_right_logical(x, jnp.int32(32 - r)),
    )


def _tf_bits(k0, k1, ctr):
    ks2 = jnp.bitwise_xor(jnp.bitwise_xor(k0, k1), jnp.int32(0x1BD11BDA))
    ks = [k0, k1, ks2]
    rot = [[13, 15, 26, 6], [17, 29, 16, 24]]
    x0 = jnp.broadcast_to(ks[0], ctr.shape)
    x1 = ctr + ks[1]
    for i in range(5):
        for r in rot[i % 2]:
            x0 = x0 + x1
            x1 = _rotl(x1, r)
            x1 = jnp.bitwise_xor(x0, x1)
        x0 = x0 + ks[(i + 1) % 3]
        x1 = x1 + ks[(i + 2) % 3] + jnp.int32(i + 1)
    return jnp.bitwise_xor(x0, x1)


def _partner(x, d, m_first):
    b, g = x.shape
    if 2 * d == g:
        return jnp.roll(x, d, axis=-1)
    if 2 * d >= 512:
        return jnp.roll(x.reshape(b, g // (2 * d), 2 * d), d, axis=-1).reshape(b, g)
    return jnp.where(m_first, jnp.roll(x, -d, axis=-1), jnp.roll(x, d, axis=-1))


def _bitonic_sort(key, val, n, iota1):
    s = 2
    while s <= n:
        d = s // 2
        while d >= 1:
            m_first = (iota1 & d) == 0
            kp = _partner(key, d, m_first)
            vp = _partner(val, d, m_first)
            gt = key > kp
            mdir = ((iota1 & d) != 0) != ((iota1 & s) != 0)
            take_p = (gt != mdir) & (key != kp)
            key = jnp.where(take_p, kp, key)
            val = jnp.where(take_p, vp, val)
            d //= 2
        s *= 2
    one = jnp.int32(1)
    zero = jnp.int32(0)
    for off in (0, 1, 0, 1):
        is_left = ((iota1 & 1) == off) & (iota1 < (n - 1))
        kr = jnp.roll(key, -1, axis=-1)
        vr = jnp.roll(val, -1, axis=-1)
        swap = (key == kr) & (val > vr) & is_left
        swap_i = jnp.where(swap, one, zero)
        swap_r = jnp.roll(swap_i, 1, axis=-1) == 1
        vl = jnp.roll(val, 1, axis=-1)
        val = jnp.where(swap, vr, jnp.where(swap_r, vl, val))
    return key, val


def _sort_body(keys_ref, x1_ref, pos_ref):
    ks = keys_ref[0, 0]
    r = ks.shape[0]
    iota2 = lax.broadcasted_iota(jnp.int32, (2 * r, G), 1)
    iota1 = lax.broadcasted_iota(jnp.int32, (1, G), 1)
    k0 = jnp.concatenate([ks[:, 0:1], ks[:, 2:3]], axis=0)
    k1 = jnp.concatenate([ks[:, 1:2], ks[:, 3:4]], axis=0)
    bits = _tf_bits(k0, k1, iota2)
    skey = jnp.bitwise_xor(bits, MSB)
    _, val = _bitonic_sort(skey, iota2, G, iota1)
    x1_ref[...] = val[:r]
    pos_ref[...] = val[r:, :CP]


def _binom_body(gv_ref, prm_ref, out_ref):
    gv = gv_ref[...]
    prm = prm_ref[0, 0]
    r = gv.shape[0]
    lognm = prm[:, 0:1]
    plt = prm[:, 1:2]
    count = jnp.floor(gv)
    row0 = pl.program_id(0) * r
    iota_c = lax.broadcasted_iota(jnp.int32, (r, CP), 1)
    iota_r = lax.broadcasted_iota(jnp.int32, (r, CP), 0)
    ctr = (row0 + iota_r) * jnp.int32(C) + iota_c
    num_geom = jnp.zeros((r, CP), jnp.float32)
    gsum = jnp.zeros((r, CP), jnp.float32)
    for t in range(BINOM_ITERS):
        k0, k1 = _BINOM_SUBKEYS[t]
        bits = _tf_bits(jnp.int32(k0), jnp.int32(k1), ctr)
        fb = jnp.bitwise_or(lax.shift_right_logical(bits, jnp.int32(9)), EXP1)
        u = lax.bitcast_convert_type(fb, jnp.float32) - jnp.float32(1.0)
        num_geom = jnp.where(gsum <= count, num_geom + 1.0, num_geom)
        geom = jnp.ceil(jnp.log(u) / lognm)
        gsum = gsum + geom
    samples = num_geom - jnp.float32(1.0)
    out_ref[...] = jnp.where(plt > 0.5, samples, count - samples)


_NW = 32
_ROWS_PER_W = N // _NW


@functools.lru_cache(maxsize=1)
def _make_sc_gather():
    mesh = plsc.VectorSubcoreMesh(core_axis_name="c", subcore_axis_name="s")

    @functools.partial(
        pl.kernel,
        mesh=mesh,
        compiler_params=pltpu.CompilerParams(needs_layout_passes=False),
        out_type=[
            jax.ShapeDtypeStruct((N, CP), jnp.int32),
            jax.ShapeDtypeStruct((N, CP), jnp.float32),
        ],
        scratch_types=[
            pltpu.VMEM((G,), jnp.int32),
            pltpu.VMEM((G,), jnp.float32),
            pltpu.VMEM((CP,), jnp.int32),
            pltpu.VMEM((CP,), jnp.int32),
            pltpu.VMEM((CP,), jnp.float32),
        ],
    )
    def _sc_gather(x1_hbm, gv_hbm, pos_hbm, perm_hbm, out_hbm,
                   x1_v, gv_v, pos_v, perm_v, val_v):
        wid = lax.axis_index("s") * 2 + lax.axis_index("c")

        def row_body(i, _):
            row = wid * _ROWS_PER_W + i
            pltpu.sync_copy(x1_hbm.at[row], x1_v)
            pltpu.sync_copy(gv_hbm.at[row], gv_v)
            pltpu.sync_copy(pos_hbm.at[row], pos_v)

            def g_body(j, _):
                pos16 = pos_v[pl.ds(j * 16, 16)]
                perm16 = plsc.load_gather(x1_v, [pos16])
                val16 = plsc.load_gather(gv_v, [perm16])
                perm_v[pl.ds(j * 16, 16)] = perm16
                val_v[pl.ds(j * 16, 16)] = val16
                return 0

            lax.fori_loop(0, CP // 16, g_body, 0)
            pltpu.sync_copy(perm_v, perm_hbm.at[row])
            pltpu.sync_copy(val_v, out_hbm.at[row])
            return 0

        lax.fori_loop(0, _ROWS_PER_W, row_body, 0)

    return _sc_gather


def _jnp_threefry(k0, k1, x0, x1):
    i32 = lambda v: jnp.asarray(v, jnp.int32)
    k0, k1, x0, x1 = i32(k0), i32(k1), i32(x0), i32(x1)
    ks = [k0, k1, k0 ^ k1 ^ jnp.int32(0x1BD11BDA)]
    rot = [[13, 15, 26, 6], [17, 29, 16, 24]]
    x0 = x0 + ks[0]
    x1 = x1 + ks[1]
    for i in range(5):
        for r in rot[i % 2]:
            x0 = x0 + x1
            x1 = _rotl(x1, r)
            x1 = x0 ^ x1
        x0 = x0 + ks[(i + 1) % 3]
        x1 = x1 + ks[(i + 2) % 3] + jnp.int32(i + 1)
    return x0, x1


def _row_sort_keys(obs_seed_n):
    seed = obs_seed_n.astype(jnp.int32)
    z = jnp.zeros_like(seed)
    rk0, rk1 = _jnp_threefry(jnp.int32(0), jnp.int32(42), z, seed)
    s10, s11 = _jnp_threefry(rk0, rk1, z, z + 1)
    c10, c11 = _jnp_threefry(rk0, rk1, z, z)
    s20, s21 = _jnp_threefry(c10, c11, z, z + 1)
    return jnp.stack([s10, s11, s20, s21], axis=-1)


SORT_R = 8
BIN_R = 64


def kernel(gene_value_ng, total_mrna_umis_ng, assay_n, cell_type_n, tissue_n,
           gene_id_g, obs_seed_n):
    n, g = gene_value_ng.shape

    keys_n4 = _row_sort_keys(obs_seed_n).reshape(n // SORT_R, 1, SORT_R, 4)

    x1_ng, pos_ncp = pl.pallas_call(
        _sort_body,
        grid=(n // SORT_R,),
        compiler_params=pltpu.CompilerParams(dimension_semantics=("parallel",)),
        in_specs=[pl.BlockSpec((1, 1, SORT_R, 4), lambda i: (i, 0, 0, 0))],
        out_specs=[
            pl.BlockSpec((SORT_R, G), lambda i: (i, 0)),
            pl.BlockSpec((SORT_R, CP), lambda i: (i, 0)),
        ],
        out_shape=[
            jax.ShapeDtypeStruct((n, G), jnp.int32),
            jax.ShapeDtypeStruct((n, CP), jnp.int32),
        ],
    )(keys_n4)

    perm_ncp, gval_ncp = _make_sc_gather()(x1_ng, gene_value_ng, pos_ncp)

    total_n1 = total_mrna_umis_ng[:, 0:1].astype(jnp.float32)
    dtot_n1 = jnp.minimum(total_n1, MAX_TOTAL_MRNA_UMIS)
    p_n1 = dtot_n1 / total_n1
    plt_n1 = (p_n1 < 0.5).astype(jnp.float32)
    q_n1 = jnp.where(p_n1 < 0.5, p_n1, 1.0 - p_n1)
    lognm_n1 = jnp.log1p(-q_n1)
    prm = jnp.concatenate([lognm_n1, plt_n1], axis=-1).reshape(n // BIN_R, 1, BIN_R, 2)

    sampled_ncp = pl.pallas_call(
        _binom_body,
        grid=(n // BIN_R,),
        compiler_params=pltpu.CompilerParams(dimension_semantics=("parallel",)),
        in_specs=[
            pl.BlockSpec((BIN_R, CP), lambda i: (i, 0)),
            pl.BlockSpec((1, 1, BIN_R, 2), lambda i: (i, 0, 0, 0)),
        ],
        out_specs=pl.BlockSpec((BIN_R, CP), lambda i: (i, 0)),
        out_shape=jax.ShapeDtypeStruct((n, CP), jnp.float32),
    )(gval_ncp, prm)

    sampled_nc = sampled_ncp[:, :C]
    gene_id_nc = perm_ncp[:, :C]
    rounded_total_nc = jnp.broadcast_to(jnp.round(dtot_n1), (n, C))
    assay_nc = jnp.broadcast_to(assay_n[:, None], (n, C)).astype(jnp.int32)
    return (
        sampled_nc,
        rounded_total_nc,
        gene_id_nc,
        assay_nc,
        cell_type_n.astype(jnp.int32),
        tissue_n.astype(jnp.int32),
    )


# speedup vs baseline: 1.2395x; 1.0629x over previous
import functools

import numpy as np
import jax
import jax.numpy as jnp
from jax import lax
from jax.experimental import pallas as pl
from jax.experimental.pallas import tpu as pltpu
from jax.experimental.pallas import tpu_sc as plsc

N = 1024
G = 16384
CONTEXT_LEN = 2048
M = 2
C = CONTEXT_LEN - M
CP = 2048
MAX_TOTAL_MRNA_UMIS = 10000.0
MSB = np.int32(np.uint32(0x80000000).view(np.int32))
EXP1 = np.int32(np.uint32(0x3F800000).view(np.int32))
BINOM_ITERS = 10


_U32 = np.uint32


def _np_threefry(k0, k1, x0, x1):
    k0, k1 = _U32(k0), _U32(k1)
    x0, x1 = _U32(x0), _U32(x1)
    ks = [k0, k1, k0 ^ k1 ^ _U32(0x1BD11BDA)]
    rot = [[13, 15, 26, 6], [17, 29, 16, 24]]
    x0 = _U32(x0 + ks[0])
    x1 = _U32(x1 + ks[1])
    for i in range(5):
        for r in rot[i % 2]:
            x0 = _U32(x0 + x1)
            x1 = _U32((_U32(x1 << _U32(r)) | _U32(x1 >> _U32(32 - r))))
            x1 = x0 ^ x1
        x0 = _U32(x0 + ks[(i + 1) % 3])
        x1 = _U32(x1 + ks[(i + 2) % 3] + _U32(i + 1))
    return x0, x1


def _np_fold(key, data):
    return _np_threefry(key[0], key[1], 0, data)


def _binom_subkeys():
    key = (_U32(0), _U32(7))
    subs = []
    with np.errstate(over="ignore"):
        for _ in range(BINOM_ITERS):
            subs.append(_np_fold(key, 0))
            key = _np_fold(key, 1)
    as_i32 = lambda v: int(np.asarray(v, np.uint32).view(np.int32))
    return [(as_i32(a), as_i32(b)) for a, b in subs]


_BINOM_SUBKEYS = _binom_subkeys()


def _rotl(x, r):
    return jnp.bitwise_or(
        lax.shift_left(x, jnp.int32(r)),
        lax.shift_right_logical(x, jnp.int32(32 - r)),
    )


def _tf_bits(k0, k1, ctr):
    ks2 = jnp.bitwise_xor(jnp.bitwise_xor(k0, k1), jnp.int32(0x1BD11BDA))
    ks = [k0, k1, ks2]
    rot = [[13, 15, 26, 6], [17, 29, 16, 24]]
    x0 = jnp.broadcast_to(ks[0], ctr.shape)
    x1 = ctr + ks[1]
    for i in range(5):
        for r in rot[i % 2]:
            x0 = x0 + x1
            x1 = _rotl(x1, r)
            x1 = jnp.bitwise_xor(x0, x1)
        x0 = x0 + ks[(i + 1) % 3]
        x1 = x1 + ks[(i + 2) % 3] + jnp.int32(i + 1)
    return jnp.bitwise_xor(x0, x1)


def _partner(x, d, m_first):
    b, g = x.shape
    if 2 * d == g:
        return jnp.roll(x, d, axis=-1)
    if 2 * d >= 512:
        return jnp.roll(x.reshape(b, g // (2 * d), 2 * d), d, axis=-1).reshape(b, g)
    return jnp.where(m_first, jnp.roll(x, -d, axis=-1), jnp.roll(x, d, axis=-1))


def _cx(key, val, iota1, sdir, d, lex):
    m_first = (iota1 & d) == 0
    kp = _partner(key, d, m_first)
    vp = _partner(val, d, m_first)
    mdir = ((iota1 & d) != 0) != ((iota1 & sdir) != 0)
    if lex:
        gt = (key > kp) | ((key == kp) & (val > vp))
        take_p = gt != mdir
    else:
        take_p = ((key > kp) != mdir) & (key != kp)
    return jnp.where(take_p, kp, key), jnp.where(take_p, vp, val)


def _sort_stages(key, val, iota1, s_lo, s_hi):
    s = s_lo
    while s <= s_hi:
        d = s // 2
        while d >= 1:
            key, val = _cx(key, val, iota1, s, d, lex=False)
            d //= 2
        s *= 2
    return key, val


def _merge_stages(key, val, iota1, blk, sdir):
    d = blk // 2
    while d >= 1:
        key, val = _cx(key, val, iota1, sdir, d, lex=True)
        d //= 2
    return key, val


def _tie_cleanup(key, val, iota1, blk, desc):
    one = jnp.int32(1)
    zero = jnp.int32(0)
    for off in (0, 1, 0, 1):
        is_left = ((iota1 & 1) == off) & ((iota1 & (blk - 1)) != (blk - 1))
        kr = jnp.roll(key, -1, axis=-1)
        vr = jnp.roll(val, -1, axis=-1)
        vgt = val > vr
        if desc is not None:
            vgt = vgt != desc
        swap = (key == kr) & vgt & is_left
        swap_i = jnp.where(swap, one, zero)
        swap_r = jnp.roll(swap_i, 1, axis=-1) == 1
        vl = jnp.roll(val, 1, axis=-1)
        val = jnp.where(swap, vr, jnp.where(swap_r, vl, val))
    return val


def _topk_halve(key, val, w):
    a_k, a_v, b_k, b_v = [], [], [], []
    for c in range(w // (2 * CP)):
        lo = c * 2 * CP
        a_k.append(key[:, lo : lo + CP])
        a_v.append(val[:, lo : lo + CP])
        b_k.append(key[:, lo + CP : lo + 2 * CP])
        b_v.append(val[:, lo + CP : lo + 2 * CP])
    cat = lambda xs: xs[0] if len(xs) == 1 else jnp.concatenate(xs, axis=-1)
    ak, av, bk, bv = cat(a_k), cat(a_v), cat(b_k), cat(b_v)
    agtb = (ak > bk) | ((ak == bk) & (av > bv))
    return jnp.where(agtb, bk, ak), jnp.where(agtb, bv, av)


def _sort_body(keys_ref, x1_ref, pos_ref):
    ks = keys_ref[0, 0]
    r = ks.shape[0]
    iota2 = lax.broadcasted_iota(jnp.int32, (2 * r, G), 1)
    iota1 = lax.broadcasted_iota(jnp.int32, (1, G), 1)
    k0 = jnp.concatenate([ks[:, 0:1], ks[:, 2:3]], axis=0)
    k1 = jnp.concatenate([ks[:, 1:2], ks[:, 3:4]], axis=0)
    bits = _tf_bits(k0, k1, iota2)
    skey = jnp.bitwise_xor(bits, MSB)
    key, val = _sort_stages(skey, iota2, iota1, 2, CP)
    key1, val1 = _sort_stages(key[:r], val[:r], iota1, 2 * CP, G)
    x1_ref[...] = _tie_cleanup(key1, val1, iota1, G, None)
    key2, val2 = key[r:], val[r:]
    descm = (iota1 & CP) != 0
    val2 = _tie_cleanup(key2, val2, iota1, CP, descm)
    w = G
    while w > CP:
        key2, val2 = _topk_halve(key2, val2, w)
        w //= 2
        iow = lax.broadcasted_iota(jnp.int32, (1, w), 1)
        key2, val2 = _merge_stages(key2, val2, iow, CP, CP if w > CP else 2 * CP)
    pos_ref[...] = val2


def _binom_body(gv_ref, prm_ref, out_ref):
    gv = gv_ref[...]
    prm = prm_ref[0, 0]
    r = gv.shape[0]
    lognm = prm[:, 0:1]
    plt = prm[:, 1:2]
    count = jnp.floor(gv)
    row0 = pl.program_id(0) * r
    iota_c = lax.broadcasted_iota(jnp.int32, (r, CP), 1)
    iota_r = lax.broadcasted_iota(jnp.int32, (r, CP), 0)
    ctr = (row0 + iota_r) * jnp.int32(C) + iota_c
    num_geom = jnp.zeros((r, CP), jnp.float32)
    gsum = jnp.zeros((r, CP), jnp.float32)
    for t in range(BINOM_ITERS):
        k0, k1 = _BINOM_SUBKEYS[t]
        bits = _tf_bits(jnp.int32(k0), jnp.int32(k1), ctr)
        fb = jnp.bitwise_or(lax.shift_right_logical(bits, jnp.int32(9)), EXP1)
        u = lax.bitcast_convert_type(fb, jnp.float32) - jnp.float32(1.0)
        num_geom = jnp.where(gsum <= count, num_geom + 1.0, num_geom)
        geom = jnp.ceil(jnp.log(u) / lognm)
        gsum = gsum + geom
    samples = num_geom - jnp.float32(1.0)
    out_ref[...] = jnp.where(plt > 0.5, samples, count - samples)


_NW = 32
_ROWS_PER_W = N // _NW


@functools.lru_cache(maxsize=1)
def _make_sc_gather():
    mesh = plsc.VectorSubcoreMesh(core_axis_name="c", subcore_axis_name="s")

    @functools.partial(
        pl.kernel,
        mesh=mesh,
        compiler_params=pltpu.CompilerParams(needs_layout_passes=False),
        out_type=[
            jax.ShapeDtypeStruct((N, CP), jnp.int32),
            jax.ShapeDtypeStruct((N, CP), jnp.float32),
        ],
        scratch_types=[
            pltpu.VMEM((G,), jnp.int32),
            pltpu.VMEM((G,), jnp.float32),
            pltpu.VMEM((CP,), jnp.int32),
            pltpu.VMEM((CP,), jnp.int32),
            pltpu.VMEM((CP,), jnp.float32),
        ],
    )
    def _sc_gather(x1_hbm, gv_hbm, pos_hbm, perm_hbm, out_hbm,
                   x1_v, gv_v, pos_v, perm_v, val_v):
        wid = lax.axis_index("s") * 2 + lax.axis_index("c")

        def row_body(i, _):
            row = wid * _ROWS_PER_W + i
            pltpu.sync_copy(x1_hbm.at[row], x1_v)
            pltpu.sync_copy(gv_hbm.at[row], gv_v)
            pltpu.sync_copy(pos_hbm.at[row], pos_v)

            def g_body(j, _):
                pos16 = pos_v[pl.ds(j * 16, 16)]
                perm16 = plsc.load_gather(x1_v, [pos16])
                val16 = plsc.load_gather(gv_v, [perm16])
                perm_v[pl.ds(j * 16, 16)] = perm16
                val_v[pl.ds(j * 16, 16)] = val16
                return 0

            lax.fori_loop(0, CP // 16, g_body, 0)
            pltpu.sync_copy(perm_v, perm_hbm.at[row])
            pltpu.sync_copy(val_v, out_hbm.at[row])
            return 0

        lax.fori_loop(0, _ROWS_PER_W, row_body, 0)

    return _sc_gather


def _jnp_threefry(k0, k1, x0, x1):
    i32 = lambda v: jnp.asarray(v, jnp.int32)
    k0, k1, x0, x1 = i32(k0), i32(k1), i32(x0), i32(x1)
    ks = [k0, k1, k0 ^ k1 ^ jnp.int32(0x1BD11BDA)]
    rot = [[13, 15, 26, 6], [17, 29, 16, 24]]
    x0 = x0 + ks[0]
    x1 = x1 + ks[1]
    for i in range(5):
        for r in rot[i % 2]:
            x0 = x0 + x1
            x1 = _rotl(x1, r)
            x1 = x0 ^ x1
        x0 = x0 + ks[(i + 1) % 3]
        x1 = x1 + ks[(i + 2) % 3] + jnp.int32(i + 1)
    return x0, x1


def _row_sort_keys(obs_seed_n):
    seed = obs_seed_n.astype(jnp.int32)
    z = jnp.zeros_like(seed)
    rk0, rk1 = _jnp_threefry(jnp.int32(0), jnp.int32(42), z, seed)
    s10, s11 = _jnp_threefry(rk0, rk1, z, z + 1)
    c10, c11 = _jnp_threefry(rk0, rk1, z, z)
    s20, s21 = _jnp_threefry(c10, c11, z, z + 1)
    return jnp.stack([s10, s11, s20, s21], axis=-1)


SORT_R = 8
BIN_R = 64


def kernel(gene_value_ng, total_mrna_umis_ng, assay_n, cell_type_n, tissue_n,
           gene_id_g, obs_seed_n):
    n, g = gene_value_ng.shape

    keys_n4 = _row_sort_keys(obs_seed_n).reshape(n // SORT_R, 1, SORT_R, 4)

    x1_ng, pos_ncp = pl.pallas_call(
        _sort_body,
        grid=(n // SORT_R,),
        compiler_params=pltpu.CompilerParams(dimension_semantics=("parallel",)),
        in_specs=[pl.BlockSpec((1, 1, SORT_R, 4), lambda i: (i, 0, 0, 0))],
        out_specs=[
            pl.BlockSpec((SORT_R, G), lambda i: (i, 0)),
            pl.BlockSpec((SORT_R, CP), lambda i: (i, 0)),
        ],
        out_shape=[
            jax.ShapeDtypeStruct((n, G), jnp.int32),
            jax.ShapeDtypeStruct((n, CP), jnp.int32),
        ],
    )(keys_n4)

    perm_ncp, gval_ncp = _make_sc_gather()(x1_ng, gene_value_ng, pos_ncp)

    total_n1 = total_mrna_umis_ng[:, 0:1].astype(jnp.float32)
    dtot_n1 = jnp.minimum(total_n1, MAX_TOTAL_MRNA_UMIS)
    p_n1 = dtot_n1 / total_n1
    plt_n1 = (p_n1 < 0.5).astype(jnp.float32)
    q_n1 = jnp.where(p_n1 < 0.5, p_n1, 1.0 - p_n1)
    lognm_n1 = jnp.log1p(-q_n1)
    prm = jnp.concatenate([lognm_n1, plt_n1], axis=-1).reshape(n // BIN_R, 1, BIN_R, 2)

    sampled_ncp = pl.pallas_call(
        _binom_body,
        grid=(n // BIN_R,),
        compiler_params=pltpu.CompilerParams(dimension_semantics=("parallel",)),
        in_specs=[
            pl.BlockSpec((BIN_R, CP), lambda i: (i, 0)),
            pl.BlockSpec((1, 1, BIN_R, 2), lambda i: (i, 0, 0, 0)),
        ],
        out_specs=pl.BlockSpec((BIN_R, CP), lambda i: (i, 0)),
        out_shape=jax.ShapeDtypeStruct((n, CP), jnp.float32),
    )(gval_ncp, prm)

    sampled_nc = sampled_ncp[:, :C]
    gene_id_nc = perm_ncp[:, :C]
    rounded_total_nc = jnp.broadcast_to(jnp.round(dtot_n1), (n, C))
    assay_nc = jnp.broadcast_to(assay_n[:, None], (n, C)).astype(jnp.int32)
    return (
        sampled_nc,
        rounded_total_nc,
        gene_id_nc,
        assay_nc,
        cell_type_n.astype(jnp.int32),
        tissue_n.astype(jnp.int32),
    )


# speedup vs baseline: 1.3135x; 1.0597x over previous
import functools

import numpy as np
import jax
import jax.numpy as jnp
from jax import lax
from jax.experimental import pallas as pl
from jax.experimental.pallas import tpu as pltpu
from jax.experimental.pallas import tpu_sc as plsc

N = 1024
G = 16384
CONTEXT_LEN = 2048
M = 2
C = CONTEXT_LEN - M
CP = 2048
MAX_TOTAL_MRNA_UMIS = 10000.0
MSB = np.int32(np.uint32(0x80000000).view(np.int32))
EXP1 = np.int32(np.uint32(0x3F800000).view(np.int32))
BINOM_ITERS = 10


_U32 = np.uint32


def _np_threefry(k0, k1, x0, x1):
    k0, k1 = _U32(k0), _U32(k1)
    x0, x1 = _U32(x0), _U32(x1)
    ks = [k0, k1, k0 ^ k1 ^ _U32(0x1BD11BDA)]
    rot = [[13, 15, 26, 6], [17, 29, 16, 24]]
    x0 = _U32(x0 + ks[0])
    x1 = _U32(x1 + ks[1])
    for i in range(5):
        for r in rot[i % 2]:
            x0 = _U32(x0 + x1)
            x1 = _U32((_U32(x1 << _U32(r)) | _U32(x1 >> _U32(32 - r))))
            x1 = x0 ^ x1
        x0 = _U32(x0 + ks[(i + 1) % 3])
        x1 = _U32(x1 + ks[(i + 2) % 3] + _U32(i + 1))
    return x0, x1


def _np_fold(key, data):
    return _np_threefry(key[0], key[1], 0, data)


def _binom_subkeys():
    key = (_U32(0), _U32(7))
    subs = []
    with np.errstate(over="ignore"):
        for _ in range(BINOM_ITERS):
            subs.append(_np_fold(key, 0))
            key = _np_fold(key, 1)
    as_i32 = lambda v: int(np.asarray(v, np.uint32).view(np.int32))
    return [(as_i32(a), as_i32(b)) for a, b in subs]


_BINOM_SUBKEYS = _binom_subkeys()


def _rotl(x, r):
    return jnp.bitwise_or(
        lax.shift_left(x, jnp.int32(r)),
        lax.shift_right_logical(x, jnp.int32(32 - r)),
    )


def _tf_bits(k0, k1, ctr):
    ks2 = jnp.bitwise_xor(jnp.bitwise_xor(k0, k1), jnp.int32(0x1BD11BDA))
    ks = [k0, k1, ks2]
    rot = [[13, 15, 26, 6], [17, 29, 16, 24]]
    x0 = jnp.broadcast_to(ks[0], ctr.shape)
    x1 = ctr + ks[1]
    for i in range(5):
        for r in rot[i % 2]:
            x0 = x0 + x1
            x1 = _rotl(x1, r)
            x1 = jnp.bitwise_xor(x0, x1)
        x0 = x0 + ks[(i + 1) % 3]
        x1 = x1 + ks[(i + 2) % 3] + jnp.int32(i + 1)
    return jnp.bitwise_xor(x0, x1)


def _partner(x, d, m_first):
    b, g = x.shape
    if 2 * d == g:
        return jnp.roll(x, d, axis=-1)
    if 2 * d >= 512:
        return jnp.roll(x.reshape(b, g // (2 * d), 2 * d), d, axis=-1).reshape(b, g)
    return jnp.where(m_first, jnp.roll(x, -d, axis=-1), jnp.roll(x, d, axis=-1))


def _cx(key, val, iota1, sdir, d, lex):
    m_first = (iota1 & d) == 0
    kp = _partner(key, d, m_first)
    vp = _partner(val, d, m_first)
    mdir = ((iota1 & d) != 0) != ((iota1 & sdir) != 0)
    if lex:
        gt = (key > kp) | ((key == kp) & (val > vp))
        take_p = gt != mdir
    else:
        take_p = ((key > kp) != mdir) & (key != kp)
    return jnp.where(take_p, kp, key), jnp.where(take_p, vp, val)


def _sort_stages(key, val, iota1, s_lo, s_hi):
    s = s_lo
    while s <= s_hi:
        d = s // 2
        while d >= 1:
            key, val = _cx(key, val, iota1, s, d, lex=False)
            d //= 2
        s *= 2
    return key, val


def _merge_stages(key, val, iota1, blk, sdir):
    d = blk // 2
    while d >= 1:
        key, val = _cx(key, val, iota1, sdir, d, lex=True)
        d //= 2
    return key, val


def _tie_cleanup(key, val, iota1, blk, desc):
    one = jnp.int32(1)
    zero = jnp.int32(0)
    for off in (0, 1, 0, 1):
        is_left = ((iota1 & 1) == off) & ((iota1 & (blk - 1)) != (blk - 1))
        kr = jnp.roll(key, -1, axis=-1)
        vr = jnp.roll(val, -1, axis=-1)
        vgt = val > vr
        if desc is not None:
            vgt = vgt != desc
        swap = (key == kr) & vgt & is_left
        swap_i = jnp.where(swap, one, zero)
        swap_r = jnp.roll(swap_i, 1, axis=-1) == 1
        vl = jnp.roll(val, 1, axis=-1)
        val = jnp.where(swap, vr, jnp.where(swap_r, vl, val))
    return val


def _topk_halve(key, val, w):
    a_k, a_v, b_k, b_v = [], [], [], []
    for c in range(w // (2 * CP)):
        lo = c * 2 * CP
        a_k.append(key[:, lo : lo + CP])
        a_v.append(val[:, lo : lo + CP])
        b_k.append(key[:, lo + CP : lo + 2 * CP])
        b_v.append(val[:, lo + CP : lo + 2 * CP])
    cat = lambda xs: xs[0] if len(xs) == 1 else jnp.concatenate(xs, axis=-1)
    ak, av, bk, bv = cat(a_k), cat(a_v), cat(b_k), cat(b_v)
    agtb = (ak > bk) | ((ak == bk) & (av > bv))
    return jnp.where(agtb, bk, ak), jnp.where(agtb, bv, av)


def _sort_body(keys_ref, x1_ref, pos_ref):
    ks = keys_ref[0, 0]
    r = ks.shape[0]
    iota2 = lax.broadcasted_iota(jnp.int32, (2 * r, G), 1)
    iota1 = lax.broadcasted_iota(jnp.int32, (1, G), 1)
    k0 = jnp.concatenate([ks[:, 0:1], ks[:, 2:3]], axis=0)
    k1 = jnp.concatenate([ks[:, 1:2], ks[:, 3:4]], axis=0)
    bits = _tf_bits(k0, k1, iota2)
    skey = jnp.bitwise_xor(bits, MSB)
    key, val = _sort_stages(skey, iota2, iota1, 2, CP)
    key1, val1 = _sort_stages(key[:r], val[:r], iota1, 2 * CP, G)
    x1_ref[...] = _tie_cleanup(key1, val1, iota1, G, None)
    key2, val2 = key[r:], val[r:]
    descm = (iota1 & CP) != 0
    val2 = _tie_cleanup(key2, val2, iota1, CP, descm)
    w = G
    while w > CP:
        key2, val2 = _topk_halve(key2, val2, w)
        w //= 2
        iow = lax.broadcasted_iota(jnp.int32, (1, w), 1)
        key2, val2 = _merge_stages(key2, val2, iow, CP, CP if w > CP else 2 * CP)
    pos_ref[...] = val2


def _binom_body(gv_ref, prm_ref, out_ref):
    gv = gv_ref[...]
    prm = prm_ref[0, 0]
    r = gv.shape[0]
    lognm = prm[:, 0:1]
    plt = prm[:, 1:2]
    count = jnp.floor(gv)
    row0 = pl.program_id(0) * r
    iota_c = lax.broadcasted_iota(jnp.int32, (r, CP), 1)
    iota_r = lax.broadcasted_iota(jnp.int32, (r, CP), 0)
    ctr = (row0 + iota_r) * jnp.int32(C) + iota_c
    num_geom = jnp.zeros((r, CP), jnp.float32)
    gsum = jnp.zeros((r, CP), jnp.float32)
    for t in range(BINOM_ITERS):
        k0, k1 = _BINOM_SUBKEYS[t]
        bits = _tf_bits(jnp.int32(k0), jnp.int32(k1), ctr)
        fb = jnp.bitwise_or(lax.shift_right_logical(bits, jnp.int32(9)), EXP1)
        u = lax.bitcast_convert_type(fb, jnp.float32) - jnp.float32(1.0)
        num_geom = jnp.where(gsum <= count, num_geom + 1.0, num_geom)
        geom = jnp.ceil(jnp.log(u) / lognm)
        gsum = gsum + geom
    samples = num_geom - jnp.float32(1.0)
    out_ref[...] = jnp.where(plt > 0.5, samples, count - samples)


_NW = 32
_ROWS_PER_W = N // _NW


@functools.lru_cache(maxsize=1)
def _make_sc_gather():
    mesh = plsc.VectorSubcoreMesh(core_axis_name="c", subcore_axis_name="s")

    @functools.partial(
        pl.kernel,
        mesh=mesh,
        compiler_params=pltpu.CompilerParams(needs_layout_passes=False),
        out_type=[
            jax.ShapeDtypeStruct((N, CP), jnp.int32),
            jax.ShapeDtypeStruct((N, CP), jnp.float32),
        ],
        scratch_types=[
            pltpu.VMEM((G,), jnp.int32),
            pltpu.VMEM((G,), jnp.float32),
            pltpu.VMEM((CP,), jnp.int32),
            pltpu.VMEM((CP,), jnp.int32),
            pltpu.VMEM((CP,), jnp.float32),
        ],
    )
    def _sc_gather(x1_hbm, gv_hbm, pos_hbm, perm_hbm, out_hbm,
                   x1_v, gv_v, pos_v, perm_v, val_v):
        wid = lax.axis_index("s") * 2 + lax.axis_index("c")

        def row_body(i, _):
            row = wid * _ROWS_PER_W + i
            pltpu.sync_copy(x1_hbm.at[row], x1_v)
            pltpu.sync_copy(gv_hbm.at[row], gv_v)
            pltpu.sync_copy(pos_hbm.at[row], pos_v)

            def g_body(j, _):
                pos16 = pos_v[pl.ds(j * 16, 16)]
                perm16 = plsc.load_gather(x1_v, [pos16])
                val16 = plsc.load_gather(gv_v, [perm16])
                perm_v[pl.ds(j * 16, 16)] = perm16
                val_v[pl.ds(j * 16, 16)] = val16
                return 0

            lax.fori_loop(0, CP // 16, g_body, 0)
            pltpu.sync_copy(perm_v, perm_hbm.at[row])
            pltpu.sync_copy(val_v, out_hbm.at[row])
            return 0

        lax.fori_loop(0, _ROWS_PER_W, row_body, 0)

    return _sc_gather


def _jnp_threefry(k0, k1, x0, x1):
    i32 = lambda v: jnp.asarray(v, jnp.int32)
    k0, k1, x0, x1 = i32(k0), i32(k1), i32(x0), i32(x1)
    ks = [k0, k1, k0 ^ k1 ^ jnp.int32(0x1BD11BDA)]
    rot = [[13, 15, 26, 6], [17, 29, 16, 24]]
    x0 = x0 + ks[0]
    x1 = x1 + ks[1]
    for i in range(5):
        for r in rot[i % 2]:
            x0 = x0 + x1
            x1 = _rotl(x1, r)
            x1 = x0 ^ x1
        x0 = x0 + ks[(i + 1) % 3]
        x1 = x1 + ks[(i + 2) % 3] + jnp.int32(i + 1)
    return x0, x1


def _row_sort_keys(obs_seed_n):
    seed = obs_seed_n.astype(jnp.int32)
    z = jnp.zeros_like(seed)
    rk0, rk1 = _jnp_threefry(jnp.int32(0), jnp.int32(42), z, seed)
    s10, s11 = _jnp_threefry(rk0, rk1, z, z + 1)
    c10, c11 = _jnp_threefry(rk0, rk1, z, z)
    s20, s21 = _jnp_threefry(c10, c11, z, z + 1)
    return jnp.stack([s10, s11, s20, s21], axis=-1)


SORT_R = 16
BIN_R = 64


def kernel(gene_value_ng, total_mrna_umis_ng, assay_n, cell_type_n, tissue_n,
           gene_id_g, obs_seed_n):
    n, g = gene_value_ng.shape

    keys_n4 = _row_sort_keys(obs_seed_n).reshape(n // SORT_R, 1, SORT_R, 4)

    x1_ng, pos_ncp = pl.pallas_call(
        _sort_body,
        grid=(n // SORT_R,),
        compiler_params=pltpu.CompilerParams(dimension_semantics=("parallel",)),
        in_specs=[pl.BlockSpec((1, 1, SORT_R, 4), lambda i: (i, 0, 0, 0))],
        out_specs=[
            pl.BlockSpec((SORT_R, G), lambda i: (i, 0)),
            pl.BlockSpec((SORT_R, CP), lambda i: (i, 0)),
        ],
        out_shape=[
            jax.ShapeDtypeStruct((n, G), jnp.int32),
            jax.ShapeDtypeStruct((n, CP), jnp.int32),
        ],
    )(keys_n4)

    perm_ncp, gval_ncp = _make_sc_gather()(x1_ng, gene_value_ng, pos_ncp)

    total_n1 = total_mrna_umis_ng[:, 0:1].astype(jnp.float32)
    dtot_n1 = jnp.minimum(total_n1, MAX_TOTAL_MRNA_UMIS)
    p_n1 = dtot_n1 / total_n1
    plt_n1 = (p_n1 < 0.5).astype(jnp.float32)
    q_n1 = jnp.where(p_n1 < 0.5, p_n1, 1.0 - p_n1)
    lognm_n1 = jnp.log1p(-q_n1)
    prm = jnp.concatenate([lognm_n1, plt_n1], axis=-1).reshape(n // BIN_R, 1, BIN_R, 2)

    sampled_ncp = pl.pallas_call(
        _binom_body,
        grid=(n // BIN_R,),
        compiler_params=pltpu.CompilerParams(dimension_semantics=("parallel",)),
        in_specs=[
            pl.BlockSpec((BIN_R, CP), lambda i: (i, 0)),
            pl.BlockSpec((1, 1, BIN_R, 2), lambda i: (i, 0, 0, 0)),
        ],
        out_specs=pl.BlockSpec((BIN_R, CP), lambda i: (i, 0)),
        out_shape=jax.ShapeDtypeStruct((n, CP), jnp.float32),
    )(gval_ncp, prm)

    sampled_nc = sampled_ncp[:, :C]
    gene_id_nc = perm_ncp[:, :C]
    rounded_total_nc = jnp.broadcast_to(jnp.round(dtot_n1), (n, C))
    assay_nc = jnp.broadcast_to(assay_n[:, None], (n, C)).astype(jnp.int32)
    return (
        sampled_nc,
        rounded_total_nc,
        gene_id_nc,
        assay_nc,
        cell_type_n.astype(jnp.int32),
        tissue_n.astype(jnp.int32),
    )


# speedup vs baseline: 1.3277x; 1.0108x over previous
import functools

import numpy as np
import jax
import jax.numpy as jnp
from jax import lax
from jax.experimental import pallas as pl
from jax.experimental.pallas import tpu as pltpu
from jax.experimental.pallas import tpu_sc as plsc

N = 1024
G = 16384
CONTEXT_LEN = 2048
M = 2
C = CONTEXT_LEN - M
CP = 2048
MAX_TOTAL_MRNA_UMIS = 10000.0
MSB = np.int32(np.uint32(0x80000000).view(np.int32))
EXP1 = np.int32(np.uint32(0x3F800000).view(np.int32))
BINOM_ITERS = 10


_U32 = np.uint32


def _np_threefry(k0, k1, x0, x1):
    k0, k1 = _U32(k0), _U32(k1)
    x0, x1 = _U32(x0), _U32(x1)
    ks = [k0, k1, k0 ^ k1 ^ _U32(0x1BD11BDA)]
    rot = [[13, 15, 26, 6], [17, 29, 16, 24]]
    x0 = _U32(x0 + ks[0])
    x1 = _U32(x1 + ks[1])
    for i in range(5):
        for r in rot[i % 2]:
            x0 = _U32(x0 + x1)
            x1 = _U32((_U32(x1 << _U32(r)) | _U32(x1 >> _U32(32 - r))))
            x1 = x0 ^ x1
        x0 = _U32(x0 + ks[(i + 1) % 3])
        x1 = _U32(x1 + ks[(i + 2) % 3] + _U32(i + 1))
    return x0, x1


def _np_fold(key, data):
    return _np_threefry(key[0], key[1], 0, data)


def _binom_subkeys():
    key = (_U32(0), _U32(7))
    subs = []
    with np.errstate(over="ignore"):
        for _ in range(BINOM_ITERS):
            subs.append(_np_fold(key, 0))
            key = _np_fold(key, 1)
    as_i32 = lambda v: int(np.asarray(v, np.uint32).view(np.int32))
    return [(as_i32(a), as_i32(b)) for a, b in subs]


_BINOM_SUBKEYS = _binom_subkeys()


def _rotl(x, r):
    return jnp.bitwise_or(
        lax.shift_left(x, jnp.int32(r)),
        lax.shift_right_logical(x, jnp.int32(32 - r)),
    )


def _tf_bits(k0, k1, ctr):
    ks2 = jnp.bitwise_xor(jnp.bitwise_xor(k0, k1), jnp.int32(0x1BD11BDA))
    ks = [k0, k1, ks2]
    rot = [[13, 15, 26, 6], [17, 29, 16, 24]]
    x0 = jnp.broadcast_to(ks[0], ctr.shape)
    x1 = ctr + ks[1]
    for i in range(5):
        for r in rot[i % 2]:
            x0 = x0 + x1
            x1 = _rotl(x1, r)
            x1 = jnp.bitwise_xor(x0, x1)
        x0 = x0 + ks[(i + 1) % 3]
        x1 = x1 + ks[(i + 2) % 3] + jnp.int32(i + 1)
    return jnp.bitwise_xor(x0, x1)


def _partner(x, d, m_first):
    b, g = x.shape
    if 2 * d == g:
        return jnp.roll(x, d, axis=-1)
    if 2 * d >= 512:
        return jnp.roll(x.reshape(b, g // (2 * d), 2 * d), d, axis=-1).reshape(b, g)
    return jnp.where(m_first, jnp.roll(x, -d, axis=-1), jnp.roll(x, d, axis=-1))


def _cx(key, val, iota1, sdir, d, lex):
    m_first = (iota1 & d) == 0
    kp = _partner(key, d, m_first)
    vp = _partner(val, d, m_first)
    mdir = ((iota1 & d) != 0) != ((iota1 & sdir) != 0)
    if lex:
        gt = (key > kp) | ((key == kp) & (val > vp))
        take_p = gt != mdir
    else:
        take_p = ((key > kp) != mdir) & (key != kp)
    return jnp.where(take_p, kp, key), jnp.where(take_p, vp, val)


def _sort_stages(key, val, iota1, s_lo, s_hi):
    s = s_lo
    while s <= s_hi:
        d = s // 2
        while d >= 1:
            key, val = _cx(key, val, iota1, s, d, lex=False)
            d //= 2
        s *= 2
    return key, val


def _merge_stages(key, val, iota1, blk, sdir):
    d = blk // 2
    while d >= 1:
        key, val = _cx(key, val, iota1, sdir, d, lex=True)
        d //= 2
    return key, val


def _tie_cleanup(key, val, iota1, blk, desc):
    one = jnp.int32(1)
    zero = jnp.int32(0)
    for off in (0, 1, 0, 1):
        is_left = ((iota1 & 1) == off) & ((iota1 & (blk - 1)) != (blk - 1))
        kr = jnp.roll(key, -1, axis=-1)
        vr = jnp.roll(val, -1, axis=-1)
        vgt = val > vr
        if desc is not None:
            vgt = vgt != desc
        swap = (key == kr) & vgt & is_left
        swap_i = jnp.where(swap, one, zero)
        swap_r = jnp.roll(swap_i, 1, axis=-1) == 1
        vl = jnp.roll(val, 1, axis=-1)
        val = jnp.where(swap, vr, jnp.where(swap_r, vl, val))
    return val


def _topk_halve(key, val, w):
    a_k, a_v, b_k, b_v = [], [], [], []
    for c in range(w // (2 * CP)):
        lo = c * 2 * CP
        a_k.append(key[:, lo : lo + CP])
        a_v.append(val[:, lo : lo + CP])
        b_k.append(key[:, lo + CP : lo + 2 * CP])
        b_v.append(val[:, lo + CP : lo + 2 * CP])
    cat = lambda xs: xs[0] if len(xs) == 1 else jnp.concatenate(xs, axis=-1)
    ak, av, bk, bv = cat(a_k), cat(a_v), cat(b_k), cat(b_v)
    agtb = (ak > bk) | ((ak == bk) & (av > bv))
    return jnp.where(agtb, bk, ak), jnp.where(agtb, bv, av)


def _sort_body(keys_ref, x1_ref, pos_ref):
    ks = keys_ref[0, 0]
    r = ks.shape[0]
    iota2 = lax.broadcasted_iota(jnp.int32, (2 * r, G), 1)
    iota1 = lax.broadcasted_iota(jnp.int32, (1, G), 1)
    k0 = jnp.concatenate([ks[:, 0:1], ks[:, 2:3]], axis=0)
    k1 = jnp.concatenate([ks[:, 1:2], ks[:, 3:4]], axis=0)
    bits = _tf_bits(k0, k1, iota2)
    skey = jnp.bitwise_xor(bits, MSB)
    key, val = _sort_stages(skey, iota2, iota1, 2, CP)
    key1, val1 = _sort_stages(key[:r], val[:r], iota1, 2 * CP, G)
    x1_ref[...] = _tie_cleanup(key1, val1, iota1, G, None)
    key2, val2 = key[r:], val[r:]
    descm = (iota1 & CP) != 0
    val2 = _tie_cleanup(key2, val2, iota1, CP, descm)
    w = G
    while w > CP:
        key2, val2 = _topk_halve(key2, val2, w)
        w //= 2
        iow = lax.broadcasted_iota(jnp.int32, (1, w), 1)
        key2, val2 = _merge_stages(key2, val2, iow, CP, CP if w > CP else 2 * CP)
    pos_ref[...] = val2


def _binom_body(gv_ref, prm_ref, out_ref):
    gv = gv_ref[...]
    prm = prm_ref[0, 0]
    r = gv.shape[0]
    lognm = prm[:, 0:1]
    plt = prm[:, 1:2]
    count = jnp.floor(gv)
    row0 = pl.program_id(0) * r
    iota_c = lax.broadcasted_iota(jnp.int32, (r, CP), 1)
    iota_r = lax.broadcasted_iota(jnp.int32, (r, CP), 0)
    ctr = (row0 + iota_r) * jnp.int32(C) + iota_c
    num_geom = jnp.zeros((r, CP), jnp.float32)
    gsum = jnp.zeros((r, CP), jnp.float32)
    for t in range(BINOM_ITERS):
        k0, k1 = _BINOM_SUBKEYS[t]
        bits = _tf_bits(jnp.int32(k0), jnp.int32(k1), ctr)
        fb = jnp.bitwise_or(lax.shift_right_logical(bits, jnp.int32(9)), EXP1)
        u = lax.bitcast_convert_type(fb, jnp.float32) - jnp.float32(1.0)
        num_geom = jnp.where(gsum <= count, num_geom + 1.0, num_geom)
        geom = jnp.ceil(jnp.log(u) / lognm)
        gsum = gsum + geom
    samples = num_geom - jnp.float32(1.0)
    out_ref[...] = jnp.where(plt > 0.5, samples, count - samples)


_NW = 32
_ROWS_PER_W = N // _NW


@functools.lru_cache(maxsize=1)
def _make_sc_gather():
    mesh = plsc.VectorSubcoreMesh(core_axis_name="c", subcore_axis_name="s")

    @functools.partial(
        pl.kernel,
        mesh=mesh,
        compiler_params=pltpu.CompilerParams(needs_layout_passes=False),
        out_type=[
            jax.ShapeDtypeStruct((N, CP), jnp.int32),
            jax.ShapeDtypeStruct((N, CP), jnp.float32),
        ],
        scratch_types=[
            pltpu.VMEM((G,), jnp.int32),
            pltpu.VMEM((G,), jnp.float32),
            pltpu.VMEM((CP,), jnp.int32),
            pltpu.VMEM((CP,), jnp.int32),
            pltpu.VMEM((CP,), jnp.float32),
        ],
    )
    def _sc_gather(x1_hbm, gv_hbm, pos_hbm, perm_hbm, out_hbm,
                   x1_v, gv_v, pos_v, perm_v, val_v):
        wid = lax.axis_index("s") * 2 + lax.axis_index("c")

        def row_body(i, _):
            row = wid * _ROWS_PER_W + i
            pltpu.sync_copy(x1_hbm.at[row], x1_v)
            pltpu.sync_copy(gv_hbm.at[row], gv_v)
            pltpu.sync_copy(pos_hbm.at[row], pos_v)

            def g_body(j, _):
                pos16 = pos_v[pl.ds(j * 16, 16)]
                perm16 = plsc.load_gather(x1_v, [pos16])
                val16 = plsc.load_gather(gv_v, [perm16])
                perm_v[pl.ds(j * 16, 16)] = perm16
                val_v[pl.ds(j * 16, 16)] = val16
                return 0

            lax.fori_loop(0, CP // 16, g_body, 0)
            pltpu.sync_copy(perm_v, perm_hbm.at[row])
            pltpu.sync_copy(val_v, out_hbm.at[row])
            return 0

        lax.fori_loop(0, _ROWS_PER_W, row_body, 0)

    return _sc_gather


def _jnp_threefry(k0, k1, x0, x1):
    i32 = lambda v: jnp.asarray(v, jnp.int32)
    k0, k1, x0, x1 = i32(k0), i32(k1), i32(x0), i32(x1)
    ks = [k0, k1, k0 ^ k1 ^ jnp.int32(0x1BD11BDA)]
    rot = [[13, 15, 26, 6], [17, 29, 16, 24]]
    x0 = x0 + ks[0]
    x1 = x1 + ks[1]
    for i in range(5):
        for r in rot[i % 2]:
            x0 = x0 + x1
            x1 = _rotl(x1, r)
            x1 = x0 ^ x1
        x0 = x0 + ks[(i + 1) % 3]
        x1 = x1 + ks[(i + 2) % 3] + jnp.int32(i + 1)
    return x0, x1


def _row_sort_keys(obs_seed_n):
    seed = obs_seed_n.astype(jnp.int32)
    z = jnp.zeros_like(seed)
    rk0, rk1 = _jnp_threefry(jnp.int32(0), jnp.int32(42), z, seed)
    s10, s11 = _jnp_threefry(rk0, rk1, z, z + 1)
    c10, c11 = _jnp_threefry(rk0, rk1, z, z)
    s20, s21 = _jnp_threefry(c10, c11, z, z + 1)
    return jnp.stack([s10, s11, s20, s21], axis=-1)


SORT_R = 32
BIN_R = 64


def kernel(gene_value_ng, total_mrna_umis_ng, assay_n, cell_type_n, tissue_n,
           gene_id_g, obs_seed_n):
    n, g = gene_value_ng.shape

    keys_n4 = _row_sort_keys(obs_seed_n).reshape(n // SORT_R, 1, SORT_R, 4)

    x1_ng, pos_ncp = pl.pallas_call(
        _sort_body,
        grid=(n // SORT_R,),
        compiler_params=pltpu.CompilerParams(dimension_semantics=("parallel",)),
        in_specs=[pl.BlockSpec((1, 1, SORT_R, 4), lambda i: (i, 0, 0, 0))],
        out_specs=[
            pl.BlockSpec((SORT_R, G), lambda i: (i, 0)),
            pl.BlockSpec((SORT_R, CP), lambda i: (i, 0)),
        ],
        out_shape=[
            jax.ShapeDtypeStruct((n, G), jnp.int32),
            jax.ShapeDtypeStruct((n, CP), jnp.int32),
        ],
    )(keys_n4)

    perm_ncp, gval_ncp = _make_sc_gather()(x1_ng, gene_value_ng, pos_ncp)

    total_n1 = total_mrna_umis_ng[:, 0:1].astype(jnp.float32)
    dtot_n1 = jnp.minimum(total_n1, MAX_TOTAL_MRNA_UMIS)
    p_n1 = dtot_n1 / total_n1
    plt_n1 = (p_n1 < 0.5).astype(jnp.float32)
    q_n1 = jnp.where(p_n1 < 0.5, p_n1, 1.0 - p_n1)
    lognm_n1 = jnp.log1p(-q_n1)
    prm = jnp.concatenate([lognm_n1, plt_n1], axis=-1).reshape(n // BIN_R, 1, BIN_R, 2)

    sampled_ncp = pl.pallas_call(
        _binom_body,
        grid=(n // BIN_R,),
        compiler_params=pltpu.CompilerParams(dimension_semantics=("parallel",)),
        in_specs=[
            pl.BlockSpec((BIN_R, CP), lambda i: (i, 0)),
            pl.BlockSpec((1, 1, BIN_R, 2), lambda i: (i, 0, 0, 0)),
        ],
        out_specs=pl.BlockSpec((BIN_R, CP), lambda i: (i, 0)),
        out_shape=jax.ShapeDtypeStruct((n, CP), jnp.float32),
    )(gval_ncp, prm)

    sampled_nc = sampled_ncp[:, :C]
    gene_id_nc = perm_ncp[:, :C]
    rounded_total_nc = jnp.broadcast_to(jnp.round(dtot_n1), (n, C))
    assay_nc = jnp.broadcast_to(assay_n[:, None], (n, C)).astype(jnp.int32)
    return (
        sampled_nc,
        rounded_total_nc,
        gene_id_nc,
        assay_nc,
        cell_type_n.astype(jnp.int32),
        tissue_n.astype(jnp.int32),
    )
